# Initial kernel scaffold; baseline (speedup 1.0000x reference)
import jax
import jax.numpy as jnp
from jax import lax
from jax.experimental import pallas as pl
from jax.experimental.pallas import tpu as pltpu
from jax.experimental.pallas import tpu_sc as plsc

N = 10000
E = 160000
IN_DIM = 128
OUT_DIM = 128
REL_DIM = 16
AUXW = 32

NC, NS = 2, 16
NW = NC * NS
EPW = E // NW
B = 125
K = EPW // B
RPS = N // NS


def _proj_body(h_ref, wn_ref, na_ref, h2_ref, h2s_ref):
    h2 = lax.dot_general(h_ref[...], wn_ref[...],
                         (((1,), (1,)), ((), ())),
                         preferred_element_type=jnp.float32)
    h2_ref[...] = h2
    h2s_ref[...] = na_ref[...] * h2


def _project(h, W_node, na2d):
    blk = 400
    grid = N // blk
    return pl.pallas_call(
        _proj_body,
        grid=(grid,),
        in_specs=[
            pl.BlockSpec((blk, IN_DIM), lambda i: (i, 0)),
            pl.BlockSpec((OUT_DIM, IN_DIM), lambda i: (0, 0)),
            pl.BlockSpec((blk, 1), lambda i: (i, 0)),
        ],
        out_specs=[
            pl.BlockSpec((blk, OUT_DIM), lambda i: (i, 0)),
            pl.BlockSpec((blk, OUT_DIM), lambda i: (i, 0)),
        ],
        out_shape=[
            jax.ShapeDtypeStruct((N, OUT_DIM), jnp.float32),
            jax.ShapeDtypeStruct((N, OUT_DIM), jnp.float32),
        ],
    )(h, W_node, na2d)


def _aux_body(ea_ref, attr_ref, aux_ref):
    ea = ea_ref[...]
    rows = ea.shape[0]
    z2 = ea * attr_ref[...]
    ones = jnp.ones((rows, 1), jnp.float32)
    pad = jnp.zeros((rows, AUXW - REL_DIM - 2), jnp.float32)
    aux_ref[...] = jnp.concatenate([z2, ea, ones, pad], axis=1)


def _aux_build(ea2d, edge_attr):
    blk = 2000
    grid = E // blk
    return pl.pallas_call(
        _aux_body,
        grid=(grid,),
        in_specs=[
            pl.BlockSpec((blk, 1), lambda i: (i, 0)),
            pl.BlockSpec((blk, REL_DIM), lambda i: (i, 0)),
        ],
        out_specs=pl.BlockSpec((blk, AUXW), lambda i: (i, 0)),
        out_shape=jax.ShapeDtypeStruct((E, AUXW), jnp.float32),
    )(ea2d, edge_attr)


def _sc_body(h2s_hbm, aux_hbm, src_hbm, dst_hbm, out1_hbm, outa_hbm,
             src_v, dst_v, row_v, aux_v, acc1, acca):
    c = lax.axis_index("c")
    s = lax.axis_index("s")
    wid = s * NC + c

    def _zero_rowv(i, _):
        def _zero_lane(j, _):
            row_v[i, pl.ds(j * 16, 16)] = jnp.zeros((16,), jnp.float32)
            return 0
        return lax.fori_loop(0, OUT_DIM // 16, _zero_lane, 0)
    lax.fori_loop(0, B, _zero_rowv, 0)

    def _zero_auxv(i, _):
        def _zero_lane(j, _):
            aux_v[i, pl.ds(j * 16, 16)] = jnp.zeros((16,), jnp.float32)
            return 0
        return lax.fori_loop(0, AUXW // 16, _zero_lane, 0)
    lax.fori_loop(0, B, _zero_auxv, 0)

    for r in range(RPS // B):
        pltpu.sync_copy(row_v, acc1.at[pl.ds(s * RPS + r * B, B)])
        pltpu.sync_copy(aux_v, acca.at[pl.ds(s * RPS + r * B, B)])
    plsc.subcore_barrier()

    pltpu.sync_copy(src_hbm.at[wid], src_v)
    pltpu.sync_copy(dst_hbm.at[wid], dst_v)

    def _chunk(j, _):
        pltpu.sync_copy(h2s_hbm.at[src_v.at[j]], row_v)
        pltpu.sync_copy(aux_hbm.at[wid, j], aux_v)
        pltpu.sync_copy(row_v, acc1.at[dst_v.at[j]], add=True)
        pltpu.sync_copy(aux_v, acca.at[dst_v.at[j]], add=True)
        return 0

    lax.fori_loop(0, K, _chunk, 0)
    plsc.subcore_barrier()

    pltpu.sync_copy(acc1.at[pl.ds(s * RPS, RPS)],
                    out1_hbm.at[c, pl.ds(s * RPS, RPS)])
    pltpu.sync_copy(acca.at[pl.ds(s * RPS, RPS)],
                    outa_hbm.at[c, pl.ds(s * RPS, RPS)])


def _sc_scatter(h2s, aux4, src3, dst3):
    mesh = plsc.VectorSubcoreMesh(core_axis_name="c", subcore_axis_name="s")
    f = pl.kernel(
        _sc_body,
        out_type=[
            jax.ShapeDtypeStruct((NC, N, OUT_DIM), jnp.float32),
            jax.ShapeDtypeStruct((NC, N, AUXW), jnp.float32),
        ],
        mesh=mesh,
        scratch_types=[
            pltpu.VMEM((K, B), jnp.int32),
            pltpu.VMEM((K, B), jnp.int32),
            pltpu.VMEM((B, OUT_DIM), jnp.float32),
            pltpu.VMEM((B, AUXW), jnp.float32),
            pltpu.VMEM_SHARED((N, OUT_DIM), jnp.float32),
            pltpu.VMEM_SHARED((N, AUXW), jnp.float32),
        ],
    )
    return f(h2s, aux4, src3, dst3)


def _comb_body(o1_ref, oa_ref, h2_ref, na_ref, wr_ref, br_ref, wa_ref,
               ba_ref, out_ref):
    g1 = o1_ref[0] + o1_ref[1]
    a = oa_ref[0] + oa_ref[1]
    g2a = a[:, :REL_DIM]
    s_att = a[:, REL_DIM:REL_DIM + 1]
    deg = a[:, REL_DIM + 1:REL_DIM + 2]
    msg2 = lax.dot_general(g2a, wr_ref[...], (((1,), (1,)), ((), ())),
                           preferred_element_type=jnp.float32)
    msg2 = msg2 + s_att * br_ref[...]
    wa = wa_ref[...]
    pre = lax.dot_general(g1, wa[:, :OUT_DIM],
                          (((1,), (1,)), ((), ())),
                          preferred_element_type=jnp.float32)
    pre += lax.dot_general(msg2, wa[:, OUT_DIM:2 * OUT_DIM],
                           (((1,), (1,)), ((), ())),
                           preferred_element_type=jnp.float32)
    h2 = h2_ref[...]
    pre += lax.dot_general(h2, wa[:, 2 * OUT_DIM:],
                           (((1,), (1,)), ((), ())),
                           preferred_element_type=jnp.float32)
    pre += ba_ref[...]
    new = na_ref[...] * jax.nn.relu(pre)
    out_ref[...] = jnp.where(deg > 0, new, h2)


def _combine(out1, outa, h2, na2d, W_rel, br2d, W_apply, ba2d):
    blk = 400
    grid = N // blk
    return pl.pallas_call(
        _comb_body,
        grid=(grid,),
        in_specs=[
            pl.BlockSpec((NC, blk, OUT_DIM), lambda i: (0, i, 0)),
            pl.BlockSpec((NC, blk, AUXW), lambda i: (0, i, 0)),
            pl.BlockSpec((blk, OUT_DIM), lambda i: (i, 0)),
            pl.BlockSpec((blk, 1), lambda i: (i, 0)),
            pl.BlockSpec((OUT_DIM, REL_DIM), lambda i: (0, 0)),
            pl.BlockSpec((1, OUT_DIM), lambda i: (0, 0)),
            pl.BlockSpec((OUT_DIM, 3 * OUT_DIM), lambda i: (0, 0)),
            pl.BlockSpec((1, OUT_DIM), lambda i: (0, 0)),
        ],
        out_specs=pl.BlockSpec((blk, OUT_DIM), lambda i: (i, 0)),
        out_shape=jax.ShapeDtypeStruct((N, OUT_DIM), jnp.float32),
    )(out1, outa, h2, na2d, W_rel, br2d, W_apply, ba2d)


def kernel(h, edge_index, edge_attr, node_att, edge_att, W_node, W_rel,
           b_rel, W_apply, b_apply):
    na2d = node_att.reshape(N, 1)
    ea2d = edge_att.reshape(E, 1)
    h2, h2s = _project(h, W_node, na2d)
    aux = _aux_build(ea2d, edge_attr)
    aux4 = aux.reshape(NW, K, B, AUXW)
    src3 = edge_index[0].reshape(NW, K, B)
    dst3 = edge_index[1].reshape(NW, K, B)
    out1, outa = _sc_scatter(h2s, aux4, src3, dst3)
    return _combine(out1, outa, h2, na2d, W_rel, b_rel.reshape(1, OUT_DIM),
                    W_apply, b_apply.reshape(1, OUT_DIM))


# speedup vs baseline: 5.7086x; 5.7086x over previous
import jax
import jax.numpy as jnp
from jax import lax
from jax.experimental import pallas as pl
from jax.experimental.pallas import tpu as pltpu
from jax.experimental.pallas import tpu_sc as plsc

N = 10000
E = 160000
IN_DIM = 128
OUT_DIM = 128
REL_DIM = 16
AUXW = 32

NC, NS = 2, 16
NW = NC * NS
EPW = E // NW
B = 125
K = EPW // B
NPAD = 10240
RPS = NPAD // NS
ZR = 128


def _proj_body(h_ref, wn_ref, na_ref, h2_ref, h2s_ref):
    h2 = lax.dot_general(h_ref[...], wn_ref[...],
                         (((1,), (1,)), ((), ())),
                         preferred_element_type=jnp.float32)
    h2_ref[...] = h2
    h2s_ref[...] = na_ref[...] * h2


def _project(h, W_node, na2d):
    blk = 400
    grid = N // blk
    return pl.pallas_call(
        _proj_body,
        grid=(grid,),
        in_specs=[
            pl.BlockSpec((blk, IN_DIM), lambda i: (i, 0)),
            pl.BlockSpec((OUT_DIM, IN_DIM), lambda i: (0, 0)),
            pl.BlockSpec((blk, 1), lambda i: (i, 0)),
        ],
        out_specs=[
            pl.BlockSpec((blk, OUT_DIM), lambda i: (i, 0)),
            pl.BlockSpec((blk, OUT_DIM), lambda i: (i, 0)),
        ],
        out_shape=[
            jax.ShapeDtypeStruct((N, OUT_DIM), jnp.float32),
            jax.ShapeDtypeStruct((N, OUT_DIM), jnp.float32),
        ],
    )(h, W_node, na2d)


def _aux_body(ea_ref, attr_ref, aux_ref):
    ea = ea_ref[...]
    rows = ea.shape[0]
    z2 = ea * attr_ref[...]
    ones = jnp.ones((rows, 1), jnp.float32)
    pad = jnp.zeros((rows, AUXW - REL_DIM - 2), jnp.float32)
    aux_ref[...] = jnp.concatenate([z2, ea, ones, pad], axis=1)


def _aux_build(ea2d, edge_attr):
    blk = 2000
    grid = E // blk
    return pl.pallas_call(
        _aux_body,
        grid=(grid,),
        in_specs=[
            pl.BlockSpec((blk, 1), lambda i: (i, 0)),
            pl.BlockSpec((blk, REL_DIM), lambda i: (i, 0)),
        ],
        out_specs=pl.BlockSpec((blk, AUXW), lambda i: (i, 0)),
        out_shape=jax.ShapeDtypeStruct((E, AUXW), jnp.float32),
    )(ea2d, edge_attr)


def _zero_vmem(buf, rows, width):
    def _zero_row(i, _):
        def _zero_lane(j, _):
            buf[i, pl.ds(j * 16, 16)] = jnp.zeros((16,), jnp.float32)
            return 0
        return lax.fori_loop(0, width // 16, _zero_lane, 0)
    lax.fori_loop(0, rows, _zero_row, 0)


def _sc_wide_body(h2s_hbm, src_hbm, dst_hbm, out1_hbm,
                  src_v, dst_v, row_v, zb1, acc1):
    c = lax.axis_index("c")
    s = lax.axis_index("s")
    wid = s * NC + c

    _zero_vmem(zb1, ZR, OUT_DIM)
    for r in range(RPS // ZR):
        pltpu.sync_copy(zb1, acc1.at[pl.ds(s * RPS + r * ZR, ZR)])
    plsc.subcore_barrier()

    pltpu.sync_copy(src_hbm.at[wid], src_v)
    pltpu.sync_copy(dst_hbm.at[wid], dst_v)

    def _chunk(j, _):
        pltpu.sync_copy(h2s_hbm.at[src_v.at[j]], row_v)
        pltpu.sync_copy(row_v, acc1.at[dst_v.at[j]], add=True)
        return 0

    lax.fori_loop(0, K, _chunk, 0)
    plsc.subcore_barrier()

    pltpu.sync_copy(acc1.at[pl.ds(s * RPS, RPS)],
                    out1_hbm.at[c, pl.ds(s * RPS, RPS)])


def _sc_aux_body(aux_hbm, dst_hbm, outa_hbm, dst_v, aux_v, zba, acca):
    c = lax.axis_index("c")
    s = lax.axis_index("s")
    wid = s * NC + c

    _zero_vmem(zba, ZR, AUXW)
    for r in range(RPS // ZR):
        pltpu.sync_copy(zba, acca.at[pl.ds(s * RPS + r * ZR, ZR)])
    plsc.subcore_barrier()

    pltpu.sync_copy(dst_hbm.at[wid], dst_v)

    def _chunk(j, _):
        pltpu.sync_copy(aux_hbm.at[wid, j], aux_v)
        pltpu.sync_copy(aux_v, acca.at[dst_v.at[j]], add=True)
        return 0

    lax.fori_loop(0, K, _chunk, 0)
    plsc.subcore_barrier()

    pltpu.sync_copy(acca.at[pl.ds(s * RPS, RPS)],
                    outa_hbm.at[c, pl.ds(s * RPS, RPS)])


def _sc_scatter(h2s, aux4, src3, dst3):
    mesh = plsc.VectorSubcoreMesh(core_axis_name="c", subcore_axis_name="s")
    f_wide = pl.kernel(
        _sc_wide_body,
        out_type=jax.ShapeDtypeStruct((NC, NPAD, OUT_DIM), jnp.float32),
        mesh=mesh,
        scratch_types=[
            pltpu.VMEM((K, B), jnp.int32),
            pltpu.VMEM((K, B), jnp.int32),
            pltpu.VMEM((B, OUT_DIM), jnp.float32),
            pltpu.VMEM((ZR, OUT_DIM), jnp.float32),
            pltpu.VMEM_SHARED((NPAD, OUT_DIM), jnp.float32),
        ],
    )
    f_aux = pl.kernel(
        _sc_aux_body,
        out_type=jax.ShapeDtypeStruct((NC, NPAD, AUXW), jnp.float32),
        mesh=mesh,
        compiler_params=pltpu.CompilerParams(use_tc_tiling_on_sc=False),
        scratch_types=[
            pltpu.VMEM((K, B), jnp.int32),
            pltpu.VMEM((B, AUXW), jnp.float32),
            pltpu.VMEM((ZR, AUXW), jnp.float32),
            pltpu.VMEM_SHARED((NPAD, AUXW), jnp.float32),
        ],
    )
    return f_wide(h2s, src3, dst3), f_aux(aux4, dst3)


def _comb_body(o1_ref, oa_ref, h2_ref, na_ref, wr_ref, br_ref, wa_ref,
               ba_ref, out_ref):
    g1 = o1_ref[0] + o1_ref[1]
    a = oa_ref[0] + oa_ref[1]
    g2a = a[:, :REL_DIM]
    s_att = a[:, REL_DIM:REL_DIM + 1]
    deg = a[:, REL_DIM + 1:REL_DIM + 2]
    msg2 = lax.dot_general(g2a, wr_ref[...], (((1,), (1,)), ((), ())),
                           preferred_element_type=jnp.float32)
    msg2 = msg2 + s_att * br_ref[...]
    wa = wa_ref[...]
    pre = lax.dot_general(g1, wa[:, :OUT_DIM],
                          (((1,), (1,)), ((), ())),
                          preferred_element_type=jnp.float32)
    pre += lax.dot_general(msg2, wa[:, OUT_DIM:2 * OUT_DIM],
                           (((1,), (1,)), ((), ())),
                           preferred_element_type=jnp.float32)
    h2 = h2_ref[...]
    pre += lax.dot_general(h2, wa[:, 2 * OUT_DIM:],
                           (((1,), (1,)), ((), ())),
                           preferred_element_type=jnp.float32)
    pre += ba_ref[...]
    new = na_ref[...] * jax.nn.relu(pre)
    out_ref[...] = jnp.where(deg > 0, new, h2)


def _combine(out1, outa, h2, na2d, W_rel, br2d, W_apply, ba2d):
    blk = 400
    grid = N // blk
    return pl.pallas_call(
        _comb_body,
        grid=(grid,),
        in_specs=[
            pl.BlockSpec((NC, blk, OUT_DIM), lambda i: (0, i, 0)),
            pl.BlockSpec((NC, blk, AUXW), lambda i: (0, i, 0)),
            pl.BlockSpec((blk, OUT_DIM), lambda i: (i, 0)),
            pl.BlockSpec((blk, 1), lambda i: (i, 0)),
            pl.BlockSpec((OUT_DIM, REL_DIM), lambda i: (0, 0)),
            pl.BlockSpec((1, OUT_DIM), lambda i: (0, 0)),
            pl.BlockSpec((OUT_DIM, 3 * OUT_DIM), lambda i: (0, 0)),
            pl.BlockSpec((1, OUT_DIM), lambda i: (0, 0)),
        ],
        out_specs=pl.BlockSpec((blk, OUT_DIM), lambda i: (i, 0)),
        out_shape=jax.ShapeDtypeStruct((N, OUT_DIM), jnp.float32),
    )(out1, outa, h2, na2d, W_rel, br2d, W_apply, ba2d)


def kernel(h, edge_index, edge_attr, node_att, edge_att, W_node, W_rel,
           b_rel, W_apply, b_apply):
    na2d = node_att.reshape(N, 1)
    ea2d = edge_att.reshape(E, 1)
    h2, h2s = _project(h, W_node, na2d)
    aux = _aux_build(ea2d, edge_attr)
    aux4 = aux.reshape(NW, K, B, AUXW)
    src3 = edge_index[0].reshape(NW, K, B)
    dst3 = edge_index[1].reshape(NW, K, B)
    out1, outa = _sc_scatter(h2s, aux4, src3, dst3)
    return _combine(out1, outa, h2, na2d, W_rel, b_rel.reshape(1, OUT_DIM),
                    W_apply, b_apply.reshape(1, OUT_DIM))


# speedup vs baseline: 8.7936x; 1.5404x over previous
import jax
import jax.numpy as jnp
from jax import lax
from jax.experimental import pallas as pl
from jax.experimental.pallas import tpu as pltpu
from jax.experimental.pallas import tpu_sc as plsc

N = 10000
E = 160000
IN_DIM = 128
OUT_DIM = 128
REL_DIM = 16
AUXW = 32

NC, NS = 2, 16
NW = NC * NS
EPW = E // NW
B = 125
K = EPW // B
NPAD = 10240
RPS = NPAD // NS
ZR = 128


def _proj_body(h_ref, wn_ref, na_ref, h2_ref, h2s_ref):
    h2 = lax.dot_general(h_ref[...], wn_ref[...],
                         (((1,), (1,)), ((), ())),
                         preferred_element_type=jnp.float32)
    h2_ref[...] = h2
    h2s_ref[...] = na_ref[...] * h2


def _project(h, W_node, na2d):
    blk = 400
    grid = N // blk
    return pl.pallas_call(
        _proj_body,
        grid=(grid,),
        in_specs=[
            pl.BlockSpec((blk, IN_DIM), lambda i: (i, 0)),
            pl.BlockSpec((OUT_DIM, IN_DIM), lambda i: (0, 0)),
            pl.BlockSpec((blk, 1), lambda i: (i, 0)),
        ],
        out_specs=[
            pl.BlockSpec((blk, OUT_DIM), lambda i: (i, 0)),
            pl.BlockSpec((blk, OUT_DIM), lambda i: (i, 0)),
        ],
        out_shape=[
            jax.ShapeDtypeStruct((N, OUT_DIM), jnp.float32),
            jax.ShapeDtypeStruct((N, OUT_DIM), jnp.float32),
        ],
    )(h, W_node, na2d)


def _aux_body(eat_ref, attrt_ref, aux_ref):
    eat = eat_ref[...]
    z2t = eat * attrt_ref[...]
    blk = eat.shape[1]
    z2 = jnp.swapaxes(z2t, 0, 1)
    ea = jnp.swapaxes(eat, 0, 1)
    ones = jnp.ones((blk, 1), jnp.float32)
    pad = jnp.zeros((blk, 128 - REL_DIM - 2), jnp.float32)
    aux_ref[...] = jnp.concatenate([z2, ea, ones, pad], axis=1)


def _aux_build(eat, attr_t):
    blk = 1280
    grid = E // blk
    return pl.pallas_call(
        _aux_body,
        grid=(grid,),
        in_specs=[
            pl.BlockSpec((1, blk), lambda i: (0, i)),
            pl.BlockSpec((REL_DIM, blk), lambda i: (0, i)),
        ],
        out_specs=pl.BlockSpec((blk, 128), lambda i: (i, 0)),
        out_shape=jax.ShapeDtypeStruct((E, 128), jnp.float32),
    )(eat, attr_t)


def _zero_vmem(buf, rows, width):
    def _zero_row(i, _):
        def _zero_lane(j, _):
            buf[i, pl.ds(j * 16, 16)] = jnp.zeros((16,), jnp.float32)
            return 0
        return lax.fori_loop(0, width // 16, _zero_lane, 0)
    lax.fori_loop(0, rows, _zero_row, 0)


def _sc_wide_body(h2s_hbm, src_hbm, dst_hbm, out1_hbm,
                  src_v, dst_v, row_v, zb1, acc1):
    c = lax.axis_index("c")
    s = lax.axis_index("s")
    wid = s * NC + c

    _zero_vmem(zb1, ZR, OUT_DIM)
    for r in range(RPS // ZR):
        pltpu.sync_copy(zb1, acc1.at[pl.ds(s * RPS + r * ZR, ZR)])
    plsc.subcore_barrier()

    pltpu.sync_copy(src_hbm.at[wid], src_v)
    pltpu.sync_copy(dst_hbm.at[wid], dst_v)

    def _chunk(j, _):
        pltpu.sync_copy(h2s_hbm.at[src_v.at[j]], row_v)
        pltpu.sync_copy(row_v, acc1.at[dst_v.at[j]], add=True)
        return 0

    lax.fori_loop(0, K, _chunk, 0)
    plsc.subcore_barrier()

    pltpu.sync_copy(acc1.at[pl.ds(s * RPS, RPS)],
                    out1_hbm.at[c, pl.ds(s * RPS, RPS)])


def _sc_aux_body(aux_hbm, dst_hbm, outa_hbm, dst_v, aux_v, zba, acca):
    c = lax.axis_index("c")
    s = lax.axis_index("s")
    wid = s * NC + c

    _zero_vmem(zba, ZR, AUXW)
    for r in range(RPS // ZR):
        pltpu.sync_copy(zba, acca.at[pl.ds(s * RPS + r * ZR, ZR)])
    plsc.subcore_barrier()

    pltpu.sync_copy(dst_hbm.at[wid], dst_v)

    def _chunk(j, _):
        base = wid * EPW + j * B
        pltpu.sync_copy(aux_hbm.at[pl.ds(base, B), pl.ds(0, AUXW)], aux_v)
        pltpu.sync_copy(aux_v, acca.at[dst_v.at[j]], add=True)
        return 0

    lax.fori_loop(0, K, _chunk, 0)
    plsc.subcore_barrier()

    pltpu.sync_copy(acca.at[pl.ds(s * RPS, RPS)],
                    outa_hbm.at[c, pl.ds(s * RPS, RPS)])


def _sc_scatter2(h2s, aux128, src3, dst3):
    mesh = plsc.VectorSubcoreMesh(core_axis_name="c", subcore_axis_name="s")
    f_wide = pl.kernel(
        _sc_wide_body,
        out_type=jax.ShapeDtypeStruct((NC, NPAD, OUT_DIM), jnp.float32),
        mesh=mesh,
        scratch_types=[
            pltpu.VMEM((K, B), jnp.int32),
            pltpu.VMEM((K, B), jnp.int32),
            pltpu.VMEM((B, OUT_DIM), jnp.float32),
            pltpu.VMEM((ZR, OUT_DIM), jnp.float32),
            pltpu.VMEM_SHARED((NPAD, OUT_DIM), jnp.float32),
        ],
    )
    f_aux = pl.kernel(
        _sc_aux_body,
        out_type=jax.ShapeDtypeStruct((NC, NPAD, AUXW), jnp.float32),
        mesh=mesh,
        compiler_params=pltpu.CompilerParams(use_tc_tiling_on_sc=False),
        scratch_types=[
            pltpu.VMEM((K, B), jnp.int32),
            pltpu.VMEM((B, AUXW), jnp.float32),
            pltpu.VMEM((ZR, AUXW), jnp.float32),
            pltpu.VMEM_SHARED((NPAD, AUXW), jnp.float32),
        ],
    )
    return f_wide(h2s, src3, dst3), f_aux(aux128, dst3)


def _comb_body(o1_ref, oa_ref, h2_ref, na_ref, wr_ref, br_ref, wa_ref,
               ba_ref, out_ref):
    g1 = o1_ref[0] + o1_ref[1]
    a = oa_ref[0] + oa_ref[1]
    g2a = a[:, :REL_DIM]
    s_att = a[:, REL_DIM:REL_DIM + 1]
    deg = a[:, REL_DIM + 1:REL_DIM + 2]
    msg2 = lax.dot_general(g2a, wr_ref[...], (((1,), (1,)), ((), ())),
                           preferred_element_type=jnp.float32)
    msg2 = msg2 + s_att * br_ref[...]
    wa = wa_ref[...]
    pre = lax.dot_general(g1, wa[:, :OUT_DIM],
                          (((1,), (1,)), ((), ())),
                          preferred_element_type=jnp.float32)
    pre += lax.dot_general(msg2, wa[:, OUT_DIM:2 * OUT_DIM],
                           (((1,), (1,)), ((), ())),
                           preferred_element_type=jnp.float32)
    h2 = h2_ref[...]
    pre += lax.dot_general(h2, wa[:, 2 * OUT_DIM:],
                           (((1,), (1,)), ((), ())),
                           preferred_element_type=jnp.float32)
    pre += ba_ref[...]
    new = na_ref[...] * jax.nn.relu(pre)
    out_ref[...] = jnp.where(deg > 0, new, h2)


def _combine(out1, outa, h2, na2d, W_rel, br2d, W_apply, ba2d):
    blk = 400
    grid = N // blk
    return pl.pallas_call(
        _comb_body,
        grid=(grid,),
        in_specs=[
            pl.BlockSpec((NC, blk, OUT_DIM), lambda i: (0, i, 0)),
            pl.BlockSpec((NC, blk, AUXW), lambda i: (0, i, 0)),
            pl.BlockSpec((blk, OUT_DIM), lambda i: (i, 0)),
            pl.BlockSpec((blk, 1), lambda i: (i, 0)),
            pl.BlockSpec((OUT_DIM, REL_DIM), lambda i: (0, 0)),
            pl.BlockSpec((1, OUT_DIM), lambda i: (0, 0)),
            pl.BlockSpec((OUT_DIM, 3 * OUT_DIM), lambda i: (0, 0)),
            pl.BlockSpec((1, OUT_DIM), lambda i: (0, 0)),
        ],
        out_specs=pl.BlockSpec((blk, OUT_DIM), lambda i: (i, 0)),
        out_shape=jax.ShapeDtypeStruct((N, OUT_DIM), jnp.float32),
    )(out1, outa, h2, na2d, W_rel, br2d, W_apply, ba2d)


def kernel(h, edge_index, edge_attr, node_att, edge_att, W_node, W_rel,
           b_rel, W_apply, b_apply):
    na2d = node_att.reshape(N, 1)
    h2, h2s = _project(h, W_node, na2d)
    aux128 = _aux_build(edge_att.reshape(1, E), edge_attr.T)
    src3 = edge_index[0].reshape(NW, K, B)
    dst3 = edge_index[1].reshape(NW, K, B)
    out1, outa = _sc_scatter2(h2s, aux128, src3, dst3)
    return _combine(out1, outa, h2, na2d, W_rel, b_rel.reshape(1, OUT_DIM),
                    W_apply, b_apply.reshape(1, OUT_DIM))
